# Initial kernel scaffold; baseline (speedup 1.0000x reference)
import jax
import jax.numpy as jnp
from jax.experimental import pallas as pl


def kernel(x, edge_index, W1, a_src1, a_dst1, b1, W2, a_src2, a_dst2, b2):
    raise NotImplementedError("write your pallas kernel here")


# speedup vs baseline: 7.1508x; 7.1508x over previous
import functools

import jax
import jax.numpy as jnp
from jax import lax
from jax.experimental import pallas as pl
from jax.experimental.pallas import tpu as pltpu
from jax.experimental.pallas import tpu_sc as plsc

N = 10000
E = 320000
D = 128
DA = 144
NC = 2
NS = 16
NH = 5120
NPH = NH + 16
EPT = E // NS
K = 80
CH = EPT // K
RPT = NH // NS
ZR = 64
NEG = 0.2
L = 16


def _tc_prep_body(x_ref, w_ref, a2_ref, h_ref, sd_ref):
    h = jnp.dot(x_ref[...], w_ref[...], preferred_element_type=jnp.float32)
    h_ref[...] = h
    sd_ref[...] = lax.dot_general(
        a2_ref[...], h, (((1,), (1,)), ((), ())),
        preferred_element_type=jnp.float32)


def _tc_prep(x, w, a2):
    return pl.pallas_call(
        _tc_prep_body,
        out_shape=[
            jax.ShapeDtypeStruct((N, D), jnp.float32),
            jax.ShapeDtypeStruct((2, N), jnp.float32),
        ],
    )(x, w, a2)


def _tc_mid_body(p_ref, b_ref, w_ref, a2_ref, h_ref, sd_ref):
    acc = jnp.concatenate([p_ref[0], p_ref[1]], axis=0)[:N]
    denom = acc[:, 128:129]
    denom = jnp.where(denom == 0.0, 1.0, denom)
    h1 = jnp.maximum(acc[:, :128] / denom + b_ref[...], 0.0)
    h2 = jnp.dot(h1, w_ref[...], preferred_element_type=jnp.float32)
    h_ref[...] = h2
    sd_ref[...] = lax.dot_general(
        a2_ref[...], h2, (((1,), (1,)), ((), ())),
        preferred_element_type=jnp.float32)


def _tc_mid(p, b, w, a2):
    return pl.pallas_call(
        _tc_mid_body,
        out_shape=[
            jax.ShapeDtypeStruct((N, D), jnp.float32),
            jax.ShapeDtypeStruct((2, N), jnp.float32),
        ],
    )(p, b, w, a2)


def _tc_fin_body(p_ref, b_ref, o_ref):
    acc = jnp.concatenate([p_ref[0], p_ref[1]], axis=0)[:N]
    denom = acc[:, 128:129]
    denom = jnp.where(denom == 0.0, 1.0, denom)
    o_ref[...] = acc[:, :128] / denom + b_ref[...]


def _tc_fin(p, b):
    return pl.pallas_call(
        _tc_fin_body,
        out_shape=jax.ShapeDtypeStruct((N, D), jnp.float32),
    )(p, b)


_mesh = plsc.VectorSubcoreMesh(core_axis_name="c", subcore_axis_name="s", num_cores=NC)


@functools.partial(
    pl.kernel,
    out_type=jax.ShapeDtypeStruct((NC, NH, DA), jnp.float32),
    mesh=_mesh,
    scratch_types=[
        pltpu.VMEM((N,), jnp.float32),
        pltpu.VMEM((N,), jnp.float32),
        pltpu.VMEM((K,), jnp.int32),
        pltpu.VMEM((K,), jnp.int32),
        pltpu.VMEM((K,), jnp.int32),
        pltpu.VMEM((K,), jnp.float32),
        pltpu.VMEM((K, D), jnp.float32),
        pltpu.VMEM((K, DA), jnp.float32),
        pltpu.VMEM((ZR, DA), jnp.float32),
        pltpu.VMEM_SHARED((NPH, DA), jnp.float32),
        pltpu.SemaphoreType.DMA,
    ],
    compiler_params=pltpu.CompilerParams(needs_layout_passes=False, use_tc_tiling_on_sc=False),
)
def _sc_edge(h_hbm, sd_hbm, src_hbm, dst_hbm, out_hbm,
             s_t, d_t, src_v, dst_v, dstm_v, w_buf, rows_g, rows_s, zbuf,
             acc, gsem):
    cid = lax.axis_index("c")
    sid = lax.axis_index("s")
    ebase = sid * EPT
    nbase = cid * NH

    pltpu.sync_copy(sd_hbm.at[0], s_t)
    pltpu.sync_copy(sd_hbm.at[1], d_t)

    def _zero_row(r, _):
        for j in range(DA // L):
            zbuf[r, pl.ds(j * L, L)] = jnp.zeros((L,), jnp.float32)
        return 0
    lax.fori_loop(0, ZR, _zero_row, 0)
    for part in range(RPT // ZR):
        pltpu.sync_copy(zbuf, acc.at[pl.ds(sid * RPT + part * ZR, ZR)])

    def _max_body(i, carry):
        ms, md = carry
        ms = jnp.maximum(ms, s_t[pl.ds(i * L, L)])
        md = jnp.maximum(md, d_t[pl.ds(i * L, L)])
        return ms, md
    ninf = jnp.full((L,), -jnp.inf, jnp.float32)
    ms, md = lax.fori_loop(0, N // L, _max_body, (ninf, ninf))
    lanes = lax.iota(jnp.int32, L)
    for sh in (8, 4, 2, 1):
        perm = lanes ^ sh
        ms = jnp.maximum(ms, ms.at[perm].get(mode="promise_in_bounds"))
        md = jnp.maximum(md, md.at[perm].get(mode="promise_in_bounds"))
    mv = ms + md
    mvec = jnp.where(mv > 0.0, mv, NEG * mv)

    onehot = jnp.where(
        lax.iota(jnp.int32, L) == 0,
        jnp.ones((L,), jnp.float32), jnp.zeros((L,), jnp.float32))

    plsc.subcore_barrier()

    nbase_v = jnp.full((L,), nbase, jnp.int32)
    trash_v = jnp.full((L,), NH, jnp.int32)

    def _chunk(c, mvec):
        off = ebase + c * K
        pltpu.sync_copy(src_hbm.at[pl.ds(off, K)], src_v)
        pltpu.sync_copy(dst_hbm.at[pl.ds(off, K)], dst_v)
        pltpu.async_copy(h_hbm.at[src_v], rows_g, gsem).wait()
        for g in range(K // L):
            si = src_v[pl.ds(g * L, L)]
            di = dst_v[pl.ds(g * L, L)]
            e = plsc.load_gather(s_t, [si]) + plsc.load_gather(d_t, [di])
            e = jnp.where(e > 0.0, e, NEG * e)
            w_buf[pl.ds(g * L, L)] = jnp.exp(e - mvec)
            dl = di - nbase_v
            ok = jnp.logical_and(dl >= 0, dl < NH)
            dstm_v[pl.ds(g * L, L)] = jnp.where(ok, dl, trash_v)
        def _scale(i, _):
            idx = jnp.full((L,), i, jnp.int32)
            wv = plsc.load_gather(w_buf, [idx])
            for j in range(D // L):
                rows_s[i, pl.ds(j * L, L)] = rows_g[i, pl.ds(j * L, L)] * wv
            rows_s[i, pl.ds(D, L)] = wv * onehot
            return 0
        lax.fori_loop(0, K, _scale, 0)
        pltpu.sync_copy(rows_s, acc.at[dstm_v], add=True)
        return mvec

    lax.fori_loop(0, CH, _chunk, mvec)

    plsc.subcore_barrier()
    pltpu.sync_copy(acc.at[pl.ds(sid * RPT, RPT)],
                    out_hbm.at[cid, pl.ds(sid * RPT, RPT)])


def kernel(x, edge_index, W1, a_src1, a_dst1, b1, W2, a_src2, a_dst2, b2):
    src = edge_index[0]
    dst = edge_index[1]
    a21 = jnp.stack([a_src1, a_dst1])
    a22 = jnp.stack([a_src2, a_dst2])

    h1, sd1 = _tc_prep(x, W1, a21)
    p1 = _sc_edge(h1, sd1, src, dst)
    h2, sd2 = _tc_mid(p1, b1.reshape(1, D), W2, a22)
    p2 = _sc_edge(h2, sd2, src, dst)
    return _tc_fin(p2, b2.reshape(1, D))


# speedup vs baseline: 11.8070x; 1.6511x over previous
import functools

import jax
import jax.numpy as jnp
from jax import lax
from jax.experimental import pallas as pl
from jax.experimental.pallas import tpu as pltpu
from jax.experimental.pallas import tpu_sc as plsc

N = 10000
E = 320000
D = 128
DA = 144
NC = 2
NS = 16
NH = 5120
NPH = NH + 16
EPT = E // NS
K = 80
CH = EPT // K
IB = 50
NB = CH // IB
RPT = NH // NS
ZR = 40
NEG = 0.2
L = 16


def _tc_prep_body(x_ref, w_ref, a2_ref, h_ref, sd_ref):
    h = jnp.dot(x_ref[...], w_ref[...], preferred_element_type=jnp.float32)
    h_ref[...] = h
    sd_ref[...] = lax.dot_general(
        a2_ref[...], h, (((1,), (1,)), ((), ())),
        preferred_element_type=jnp.float32)


def _tc_prep(x, w, a2):
    return pl.pallas_call(
        _tc_prep_body,
        out_shape=[
            jax.ShapeDtypeStruct((N, D), jnp.float32),
            jax.ShapeDtypeStruct((2, N), jnp.float32),
        ],
    )(x, w, a2)


def _tc_mid_body(p_ref, b_ref, w_ref, a2_ref, h_ref, sd_ref):
    acc = jnp.concatenate([p_ref[0], p_ref[1]], axis=0)[:N]
    denom = acc[:, 128:129]
    denom = jnp.where(denom == 0.0, 1.0, denom)
    h1 = jnp.maximum(acc[:, :128] / denom + b_ref[...], 0.0)
    h2 = jnp.dot(h1, w_ref[...], preferred_element_type=jnp.float32)
    h_ref[...] = h2
    sd_ref[...] = lax.dot_general(
        a2_ref[...], h2, (((1,), (1,)), ((), ())),
        preferred_element_type=jnp.float32)


def _tc_mid(p, b, w, a2):
    return pl.pallas_call(
        _tc_mid_body,
        out_shape=[
            jax.ShapeDtypeStruct((N, D), jnp.float32),
            jax.ShapeDtypeStruct((2, N), jnp.float32),
        ],
    )(p, b, w, a2)


def _tc_fin_body(p_ref, b_ref, o_ref):
    acc = jnp.concatenate([p_ref[0], p_ref[1]], axis=0)[:N]
    denom = acc[:, 128:129]
    denom = jnp.where(denom == 0.0, 1.0, denom)
    o_ref[...] = acc[:, :128] / denom + b_ref[...]


def _tc_fin(p, b):
    return pl.pallas_call(
        _tc_fin_body,
        out_shape=jax.ShapeDtypeStruct((N, D), jnp.float32),
    )(p, b)


_mesh = plsc.VectorSubcoreMesh(core_axis_name="c", subcore_axis_name="s", num_cores=NC)


@functools.partial(
    pl.kernel,
    out_type=jax.ShapeDtypeStruct((NC, NH, DA), jnp.float32),
    mesh=_mesh,
    scratch_types=[
        pltpu.VMEM((N,), jnp.float32),
        pltpu.VMEM((N,), jnp.float32),
        pltpu.VMEM((IB, K), jnp.int32),
        pltpu.VMEM((IB, K), jnp.int32),
        pltpu.VMEM((K,), jnp.int32),
        pltpu.VMEM((K,), jnp.int32),
        pltpu.VMEM((K,), jnp.float32),
        pltpu.VMEM((K, D), jnp.float32),
        pltpu.VMEM((K, D), jnp.float32),
        pltpu.VMEM((K, DA), jnp.float32),
        pltpu.VMEM((K, DA), jnp.float32),
        pltpu.VMEM((ZR, DA), jnp.float32),
        pltpu.VMEM_SHARED((NPH, DA), jnp.float32),
        pltpu.SemaphoreType.DMA,
        pltpu.SemaphoreType.DMA,
        pltpu.SemaphoreType.DMA,
        pltpu.SemaphoreType.DMA,
    ],
    compiler_params=pltpu.CompilerParams(needs_layout_passes=False, use_tc_tiling_on_sc=False),
)
def _sc_edge(h_hbm, sd_hbm, src_hbm, dst_hbm, out_hbm,
             s_t, d_t, src_blk, dst_blk, dstm0, dstm1, w_buf,
             rows_g0, rows_g1, rows_s0, rows_s1, zbuf,
             acc, gsem0, gsem1, ssem0, ssem1):
    cid = lax.axis_index("c")
    sid = lax.axis_index("s")
    nbase = cid * NH
    dstm = (dstm0, dstm1)
    rows_g = (rows_g0, rows_g1)
    rows_s = (rows_s0, rows_s1)
    gsem = (gsem0, gsem1)
    ssem = (ssem0, ssem1)

    pltpu.sync_copy(sd_hbm.at[0], s_t)
    pltpu.sync_copy(sd_hbm.at[1], d_t)

    def _zero_row(r, _):
        for j in range(DA // L):
            zbuf[r, pl.ds(j * L, L)] = jnp.zeros((L,), jnp.float32)
        return 0
    lax.fori_loop(0, ZR, _zero_row, 0)
    for part in range(RPT // ZR):
        pltpu.sync_copy(zbuf, acc.at[pl.ds(sid * RPT + part * ZR, ZR)])

    def _max_body(i, carry):
        ms, md = carry
        ms = jnp.maximum(ms, s_t[pl.ds(i * L, L)])
        md = jnp.maximum(md, d_t[pl.ds(i * L, L)])
        return ms, md
    ninf = jnp.full((L,), -jnp.inf, jnp.float32)
    ms, md = lax.fori_loop(0, N // L, _max_body, (ninf, ninf))
    lanes = lax.iota(jnp.int32, L)
    for sh in (8, 4, 2, 1):
        perm = lanes ^ sh
        ms = jnp.maximum(ms, ms.at[perm].get(mode="promise_in_bounds"))
        md = jnp.maximum(md, md.at[perm].get(mode="promise_in_bounds"))
    mv = ms + md
    mvec = jnp.where(mv > 0.0, mv, NEG * mv)

    onehot = jnp.where(
        lax.iota(jnp.int32, L) == 0,
        jnp.ones((L,), jnp.float32), jnp.zeros((L,), jnp.float32))

    plsc.subcore_barrier()

    nbase_v = jnp.full((L,), nbase, jnp.int32)
    trash_v = jnp.full((L,), NH, jnp.int32)

    def _block(blk, mvec):
        cbase = blk * IB
        pltpu.sync_copy(src_hbm.at[sid, pl.ds(cbase, IB)], src_blk)
        pltpu.sync_copy(dst_hbm.at[sid, pl.ds(cbase, IB)], dst_blk)

        for b in range(2):
            pltpu.async_copy(h_hbm.at[src_blk.at[b]], rows_g[b], gsem[b])

        def _pair(g, mvec):
            for b in range(2):
                c = 2 * g + b
                pltpu.make_async_copy(h_hbm.at[src_blk.at[c]], rows_g[b],
                                      gsem[b]).wait()
                for q in range(K // L):
                    si = src_blk[c, pl.ds(q * L, L)]
                    di = dst_blk[c, pl.ds(q * L, L)]
                    e = plsc.load_gather(s_t, [si]) + plsc.load_gather(d_t, [di])
                    e = jnp.where(e > 0.0, e, NEG * e)
                    w_buf[pl.ds(q * L, L)] = jnp.exp(e - mvec)
                @pl.when(g >= 1)
                def _():
                    pltpu.make_async_copy(rows_s[b], acc.at[dstm[b]],
                                          ssem[b]).wait()
                for q in range(K // L):
                    di = dst_blk[c, pl.ds(q * L, L)]
                    dl = di - nbase_v
                    ok = jnp.logical_and(dl >= 0, dl < NH)
                    dstm[b][pl.ds(q * L, L)] = jnp.where(ok, dl, trash_v)
                def _scale(i, _):
                    idx = jnp.full((L,), i, jnp.int32)
                    wv = plsc.load_gather(w_buf, [idx])
                    for j in range(D // L):
                        rows_s[b][i, pl.ds(j * L, L)] = rows_g[b][i, pl.ds(j * L, L)] * wv
                    rows_s[b][i, pl.ds(D, L)] = wv * onehot
                    return 0
                lax.fori_loop(0, K, _scale, 0)
                pltpu.async_copy(rows_s[b], acc.at[dstm[b]], ssem[b], add=True)
                @pl.when(c + 2 < IB)
                def _():
                    pltpu.async_copy(h_hbm.at[src_blk.at[c + 2]], rows_g[b],
                                     gsem[b])
            return mvec

        mvec = lax.fori_loop(0, IB // 2, _pair, mvec)

        for b in range(2):
            pltpu.make_async_copy(rows_s[b], acc.at[dstm[b]], ssem[b]).wait()
        return mvec

    lax.fori_loop(0, NB, _block, mvec)

    plsc.subcore_barrier()
    pltpu.sync_copy(acc.at[pl.ds(sid * RPT, RPT)],
                    out_hbm.at[cid, pl.ds(sid * RPT, RPT)])


def kernel(x, edge_index, W1, a_src1, a_dst1, b1, W2, a_src2, a_dst2, b2):
    src = edge_index[0].reshape(NS, CH, K)
    dst = edge_index[1].reshape(NS, CH, K)
    a21 = jnp.stack([a_src1, a_dst1])
    a22 = jnp.stack([a_src2, a_dst2])

    h1, sd1 = _tc_prep(x, W1, a21)
    p1 = _sc_edge(h1, sd1, src, dst)
    h2, sd2 = _tc_mid(p1, b1.reshape(1, D), W2, a22)
    p2 = _sc_edge(h2, sd2, src, dst)
    return _tc_fin(p2, b2.reshape(1, D))


# speedup vs baseline: 20.0458x; 1.6978x over previous
import functools

import jax
import jax.numpy as jnp
from jax import lax
from jax.experimental import pallas as pl
from jax.experimental.pallas import tpu as pltpu
from jax.experimental.pallas import tpu_sc as plsc

N = 10000
E = 320000
D = 128
DW = 64
DA = 80
NC = 2
NS = 16
NP = 10240
EPT = E // NS
K = 80
CH = EPT // K
IB = 50
NB = CH // IB
RPT = NP // NS
ZR = 40
NEG = 0.2
L = 16


def _split_h(h, hp_ref):
    hp_ref[0] = h[:, :DW]
    hp_ref[1] = h[:, DW:]


def _tc_prep_body(x_ref, w_ref, a2_ref, hp_ref, sd_ref):
    h = jnp.dot(x_ref[...], w_ref[...], preferred_element_type=jnp.float32)
    _split_h(h, hp_ref)
    sd_ref[...] = lax.dot_general(
        a2_ref[...], h, (((1,), (1,)), ((), ())),
        preferred_element_type=jnp.float32)


def _tc_prep(x, w, a2):
    return pl.pallas_call(
        _tc_prep_body,
        out_shape=[
            jax.ShapeDtypeStruct((NC, N, DW), jnp.float32),
            jax.ShapeDtypeStruct((2, N), jnp.float32),
        ],
    )(x, w, a2)


def _combine(p_ref, b_ref):
    feat = jnp.concatenate([p_ref[0, :N, :DW], p_ref[1, :N, :DW]], axis=1)
    denom = p_ref[0, :N, DW:DW + 1]
    denom = jnp.where(denom == 0.0, 1.0, denom)
    return feat / denom + b_ref[...]


def _tc_mid_body(p_ref, b_ref, w_ref, a2_ref, hp_ref, sd_ref):
    h1 = jnp.maximum(_combine(p_ref, b_ref), 0.0)
    h2 = jnp.dot(h1, w_ref[...], preferred_element_type=jnp.float32)
    _split_h(h2, hp_ref)
    sd_ref[...] = lax.dot_general(
        a2_ref[...], h2, (((1,), (1,)), ((), ())),
        preferred_element_type=jnp.float32)


def _tc_mid(p, b, w, a2):
    return pl.pallas_call(
        _tc_mid_body,
        out_shape=[
            jax.ShapeDtypeStruct((NC, N, DW), jnp.float32),
            jax.ShapeDtypeStruct((2, N), jnp.float32),
        ],
    )(p, b, w, a2)


def _tc_fin_body(p_ref, b_ref, o_ref):
    o_ref[...] = _combine(p_ref, b_ref)


def _tc_fin(p, b):
    return pl.pallas_call(
        _tc_fin_body,
        out_shape=jax.ShapeDtypeStruct((N, D), jnp.float32),
    )(p, b)


_mesh = plsc.VectorSubcoreMesh(core_axis_name="c", subcore_axis_name="s", num_cores=NC)


@functools.partial(
    pl.kernel,
    out_type=jax.ShapeDtypeStruct((NC, NP, DA), jnp.float32),
    mesh=_mesh,
    scratch_types=[
        pltpu.VMEM((N,), jnp.float32),
        pltpu.VMEM((N,), jnp.float32),
        pltpu.VMEM((IB, K), jnp.int32),
        pltpu.VMEM((IB, K), jnp.int32),
        pltpu.VMEM((IB, K), jnp.int32),
        pltpu.VMEM((K,), jnp.int32),
        pltpu.VMEM((K,), jnp.int32),
        pltpu.VMEM((K,), jnp.float32),
        pltpu.VMEM((K, DW), jnp.float32),
        pltpu.VMEM((K, DW), jnp.float32),
        pltpu.VMEM((K, DA), jnp.float32),
        pltpu.VMEM((K, DA), jnp.float32),
        pltpu.VMEM((ZR, DA), jnp.float32),
        pltpu.VMEM_SHARED((NP, DA), jnp.float32),
        pltpu.SemaphoreType.DMA,
        pltpu.SemaphoreType.DMA,
        pltpu.SemaphoreType.DMA,
        pltpu.SemaphoreType.DMA,
    ],
    compiler_params=pltpu.CompilerParams(needs_layout_passes=False, use_tc_tiling_on_sc=False),
)
def _sc_edge(hp_hbm, sd_hbm, src_hbm, dst_hbm, out_hbm,
             s_t, d_t, src_blk, dst_blk, srcg_blk, dstm0, dstm1, w_buf,
             rows_g0, rows_g1, rows_s0, rows_s1, zbuf,
             acc, gsem0, gsem1, ssem0, ssem1):
    cid = lax.axis_index("c")
    sid = lax.axis_index("s")
    dstm = (dstm0, dstm1)
    rows_g = (rows_g0, rows_g1)
    rows_s = (rows_s0, rows_s1)
    gsem = (gsem0, gsem1)
    ssem = (ssem0, ssem1)

    pltpu.sync_copy(sd_hbm.at[0], s_t)
    pltpu.sync_copy(sd_hbm.at[1], d_t)

    def _zero_row(r, _):
        for j in range(DA // L):
            zbuf[r, pl.ds(j * L, L)] = jnp.zeros((L,), jnp.float32)
        return 0
    lax.fori_loop(0, ZR, _zero_row, 0)
    for part in range(RPT // ZR):
        pltpu.sync_copy(zbuf, acc.at[pl.ds(sid * RPT + part * ZR, ZR)])

    def _max_body(i, carry):
        ms, md = carry
        ms = jnp.maximum(ms, s_t[pl.ds(i * L, L)])
        md = jnp.maximum(md, d_t[pl.ds(i * L, L)])
        return ms, md
    ninf = jnp.full((L,), -jnp.inf, jnp.float32)
    ms, md = lax.fori_loop(0, N // L, _max_body, (ninf, ninf))
    lanes = lax.iota(jnp.int32, L)
    for sh in (8, 4, 2, 1):
        perm = lanes ^ sh
        ms = jnp.maximum(ms, ms.at[perm].get(mode="promise_in_bounds"))
        md = jnp.maximum(md, md.at[perm].get(mode="promise_in_bounds"))
    mv = ms + md
    mvec = jnp.where(mv > 0.0, mv, NEG * mv)

    onehot = jnp.where(
        lax.iota(jnp.int32, L) == 0,
        jnp.ones((L,), jnp.float32), jnp.zeros((L,), jnp.float32))

    plsc.subcore_barrier()

    goff_v = jnp.full((L,), cid * N, jnp.int32)

    def _block(blk, mvec):
        cbase = blk * IB
        pltpu.sync_copy(src_hbm.at[sid, pl.ds(cbase, IB)], src_blk)
        pltpu.sync_copy(dst_hbm.at[sid, pl.ds(cbase, IB)], dst_blk)

        def _goff(r, _):
            for q in range(K // L):
                srcg_blk[r, pl.ds(q * L, L)] = (
                    src_blk[r, pl.ds(q * L, L)] + goff_v)
            return 0
        lax.fori_loop(0, IB, _goff, 0)

        for b in range(2):
            pltpu.async_copy(hp_hbm.at[srcg_blk.at[b]], rows_g[b], gsem[b])

        def _pair(g, mvec):
            for b in range(2):
                c = 2 * g + b
                pltpu.make_async_copy(hp_hbm.at[srcg_blk.at[c]], rows_g[b],
                                      gsem[b]).wait()
                for q in range(K // L):
                    si = src_blk[c, pl.ds(q * L, L)]
                    di = dst_blk[c, pl.ds(q * L, L)]
                    e = plsc.load_gather(s_t, [si]) + plsc.load_gather(d_t, [di])
                    e = jnp.where(e > 0.0, e, NEG * e)
                    w_buf[pl.ds(q * L, L)] = jnp.exp(e - mvec)
                @pl.when(g >= 1)
                def _():
                    pltpu.make_async_copy(rows_s[b], acc.at[dstm[b]],
                                          ssem[b]).wait()
                for q in range(K // L):
                    dstm[b][pl.ds(q * L, L)] = dst_blk[c, pl.ds(q * L, L)]
                def _scale(i, _):
                    idx = jnp.full((L,), i, jnp.int32)
                    wv = plsc.load_gather(w_buf, [idx])
                    for j in range(DW // L):
                        rows_s[b][i, pl.ds(j * L, L)] = rows_g[b][i, pl.ds(j * L, L)] * wv
                    rows_s[b][i, pl.ds(DW, L)] = wv * onehot
                    return 0
                lax.fori_loop(0, K, _scale, 0)
                pltpu.async_copy(rows_s[b], acc.at[dstm[b]], ssem[b], add=True)
                @pl.when(c + 2 < IB)
                def _():
                    pltpu.async_copy(hp_hbm.at[srcg_blk.at[c + 2]], rows_g[b],
                                     gsem[b])
            return mvec

        mvec = lax.fori_loop(0, IB // 2, _pair, mvec)

        for b in range(2):
            pltpu.make_async_copy(rows_s[b], acc.at[dstm[b]], ssem[b]).wait()
        return mvec

    lax.fori_loop(0, NB, _block, mvec)

    plsc.subcore_barrier()
    pltpu.sync_copy(acc.at[pl.ds(sid * RPT, RPT)],
                    out_hbm.at[cid, pl.ds(sid * RPT, RPT)])


def kernel(x, edge_index, W1, a_src1, a_dst1, b1, W2, a_src2, a_dst2, b2):
    src = edge_index[0].reshape(NS, CH, K)
    dst = edge_index[1].reshape(NS, CH, K)
    a21 = jnp.stack([a_src1, a_dst1])
    a22 = jnp.stack([a_src2, a_dst2])

    hp1, sd1 = _tc_prep(x, W1, a21)
    p1 = _sc_edge(hp1.reshape(NC * N, DW), sd1, src, dst)
    hp2, sd2 = _tc_mid(p1, b1.reshape(1, D), W2, a22)
    p2 = _sc_edge(hp2.reshape(NC * N, DW), sd2, src, dst)
    return _tc_fin(p2, b2.reshape(1, D))


# speedup vs baseline: 24.3805x; 1.2162x over previous
import functools

import jax
import jax.numpy as jnp
from jax import lax
from jax.experimental import pallas as pl
from jax.experimental.pallas import tpu as pltpu
from jax.experimental.pallas import tpu_sc as plsc

N = 10000
E = 320000
D = 128
DP = 64
DN = 16
NC = 2
NS = 16
NP = 10240
EPT = E // (NC * NS)
K = 80
CH = EPT // K
RPT = NP // NS
ZR = 40
NEG = 0.2
L = 16


def _pack_h(h, hp_ref):
    hp_ref[...] = h.astype(jnp.bfloat16)


def _tc_prep_body(x_ref, w_ref, a2_ref, hp_ref, sd_ref):
    h = jnp.dot(x_ref[...], w_ref[...], preferred_element_type=jnp.float32)
    _pack_h(h, hp_ref)
    sd_ref[...] = lax.dot_general(
        a2_ref[...], h, (((1,), (1,)), ((), ())),
        preferred_element_type=jnp.float32)


def _tc_prep(x, w, a2):
    return pl.pallas_call(
        _tc_prep_body,
        out_shape=[
            jax.ShapeDtypeStruct((N, D), jnp.bfloat16),
            jax.ShapeDtypeStruct((2, N), jnp.float32),
        ],
    )(x, w, a2)


def _combine(p_ref, pd_ref, b_ref):
    feat = (p_ref[0, :N].astype(jnp.float32)
            + p_ref[1, :N].astype(jnp.float32))
    denom = pd_ref[0, :N, 0:1] + pd_ref[1, :N, 0:1]
    denom = jnp.where(denom == 0.0, 1.0, denom)
    return feat / denom + b_ref[...]


def _tc_mid_body(p_ref, pd_ref, b_ref, w_ref, a2_ref, hp_ref, sd_ref):
    h1 = jnp.maximum(_combine(p_ref, pd_ref, b_ref), 0.0)
    h2 = jnp.dot(h1, w_ref[...], preferred_element_type=jnp.float32)
    _pack_h(h2, hp_ref)
    sd_ref[...] = lax.dot_general(
        a2_ref[...], h2, (((1,), (1,)), ((), ())),
        preferred_element_type=jnp.float32)


def _tc_mid(p, pd, b, w, a2):
    return pl.pallas_call(
        _tc_mid_body,
        out_shape=[
            jax.ShapeDtypeStruct((N, D), jnp.bfloat16),
            jax.ShapeDtypeStruct((2, N), jnp.float32),
        ],
    )(p, pd, b, w, a2)


def _tc_fin_body(p_ref, pd_ref, b_ref, o_ref):
    o_ref[...] = _combine(p_ref, pd_ref, b_ref)


def _tc_fin(p, pd, b):
    return pl.pallas_call(
        _tc_fin_body,
        out_shape=jax.ShapeDtypeStruct((N, D), jnp.float32),
    )(p, pd, b)


_mesh = plsc.VectorSubcoreMesh(core_axis_name="c", subcore_axis_name="s", num_cores=NC)


@functools.partial(
    pl.kernel,
    out_type=[
        jax.ShapeDtypeStruct((NC, NP, D), jnp.bfloat16),
        jax.ShapeDtypeStruct((NC, NP, DN), jnp.float32),
    ],
    mesh=_mesh,
    scratch_types=[
        pltpu.VMEM((N,), jnp.float32),
        pltpu.VMEM((N,), jnp.float32),
        pltpu.VMEM((CH, K), jnp.int32),
        pltpu.VMEM((CH, K), jnp.int32),
        pltpu.VMEM((K,), jnp.int32),
        pltpu.VMEM((K,), jnp.int32),
        pltpu.VMEM((K,), jnp.float32),
        pltpu.VMEM((K, DP), jnp.int32),
        pltpu.VMEM((K, DP), jnp.int32),
        pltpu.VMEM((K, D), jnp.bfloat16),
        pltpu.VMEM((K, D), jnp.bfloat16),
        pltpu.VMEM((K, DN), jnp.float32),
        pltpu.VMEM((K, DN), jnp.float32),
        pltpu.VMEM((ZR, D), jnp.bfloat16),
        pltpu.VMEM((ZR, DN), jnp.float32),
        pltpu.VMEM_SHARED((NP, D), jnp.bfloat16),
        pltpu.VMEM_SHARED((NP, DN), jnp.float32),
        pltpu.SemaphoreType.DMA,
        pltpu.SemaphoreType.DMA,
        pltpu.SemaphoreType.DMA,
        pltpu.SemaphoreType.DMA,
        pltpu.SemaphoreType.DMA,
        pltpu.SemaphoreType.DMA,
    ],
    compiler_params=pltpu.CompilerParams(needs_layout_passes=False, use_tc_tiling_on_sc=False),
)
def _sc_edge(hp_hbm, sd_hbm, src_hbm, dst_hbm, out_hbm, outd_hbm,
             s_t, d_t, src_all, dst_all, dstm0, dstm1, w_buf,
             rows_g0, rows_g1, rows_s0, rows_s1, rows_w0, rows_w1,
             zbuf, zbuf_d, acc, acc_d,
             gsem0, gsem1, ssem0, ssem1, dsem0, dsem1):
    cid = lax.axis_index("c")
    sid = lax.axis_index("s")
    dstm = (dstm0, dstm1)
    rows_g = (rows_g0, rows_g1)
    rows_s = (rows_s0, rows_s1)
    rows_w = (rows_w0, rows_w1)
    gsem = (gsem0, gsem1)
    ssem = (ssem0, ssem1)
    dsem = (dsem0, dsem1)

    pltpu.sync_copy(sd_hbm.at[0], s_t)
    pltpu.sync_copy(sd_hbm.at[1], d_t)
    pltpu.sync_copy(src_hbm.at[cid, sid], src_all)
    pltpu.sync_copy(dst_hbm.at[cid, sid], dst_all)

    zb16 = jnp.zeros((2 * L,), jnp.bfloat16)
    zf = jnp.zeros((L,), jnp.float32)

    def _zero_row(r, _):
        for j in range(D // (2 * L)):
            zbuf[r, pl.ds(j * 2 * L, 2 * L)] = zb16
        return 0
    lax.fori_loop(0, ZR, _zero_row, 0)

    def _zero_drow(r, _):
        zbuf_d[r, pl.ds(0, DN)] = zf
        return 0
    lax.fori_loop(0, ZR, _zero_drow, 0)
    for part in range(RPT // ZR):
        pltpu.sync_copy(zbuf, acc.at[pl.ds(sid * RPT + part * ZR, ZR)])
        pltpu.sync_copy(zbuf_d, acc_d.at[pl.ds(sid * RPT + part * ZR, ZR)])

    def _max_body(i, carry):
        ms, md = carry
        ms = jnp.maximum(ms, s_t[pl.ds(i * L, L)])
        md = jnp.maximum(md, d_t[pl.ds(i * L, L)])
        return ms, md
    ninf = jnp.full((L,), -jnp.inf, jnp.float32)
    ms, md = lax.fori_loop(0, N // L, _max_body, (ninf, ninf))
    lanes = lax.iota(jnp.int32, L)
    for sh in (8, 4, 2, 1):
        perm = lanes ^ sh
        ms = jnp.maximum(ms, ms.at[perm].get(mode="promise_in_bounds"))
        md = jnp.maximum(md, md.at[perm].get(mode="promise_in_bounds"))
    mv = ms + md
    mvec = jnp.where(mv > 0.0, mv, NEG * mv)

    onehot = jnp.where(
        lanes == 0,
        jnp.ones((L,), jnp.float32), jnp.zeros((L,), jnp.float32))

    plsc.subcore_barrier()

    def _chunk(c, b, first, mvec):
        pltpu.make_async_copy(hp_hbm.at[src_all.at[c]], rows_g[b],
                              gsem[b]).wait()
        for q in range(K // L):
            si = src_all[c, pl.ds(q * L, L)]
            di = dst_all[c, pl.ds(q * L, L)]
            e = plsc.load_gather(s_t, [si]) + plsc.load_gather(d_t, [di])
            e = jnp.where(e > 0.0, e, NEG * e)
            w_buf[pl.ds(q * L, L)] = jnp.exp(e - mvec)
        if not first:
            pltpu.make_async_copy(rows_s[b], acc.at[dstm[b]], ssem[b]).wait()
            pltpu.make_async_copy(rows_w[b], acc_d.at[dstm[b]], dsem[b]).wait()
        for q in range(K // L):
            dstm[b][pl.ds(q * L, L)] = dst_all[c, pl.ds(q * L, L)]
        def _scale(i, _):
            idx = jnp.full((L,), i, jnp.int32)
            wv = plsc.load_gather(w_buf, [idx])
            rows_w[b][i, pl.ds(0, DN)] = wv * onehot
            for j in range(DP // L):
                words = rows_g[b][i, pl.ds(j * L, L)]
                pair = plsc.unpack(plsc.bitcast(words, jnp.bfloat16),
                                   format=plsc.PackFormat.INTERLEAVED)
                lo = pair[0].astype(jnp.float32) * wv
                hi = pair[1].astype(jnp.float32) * wv
                rows_s[b][i, pl.ds(j * 2 * L, 2 * L)] = plsc.pack(
                    lo, hi, format=plsc.PackFormat.INTERLEAVED)
            return 0
        lax.fori_loop(0, K, _scale, 0)
        pltpu.async_copy(rows_s[b], acc.at[dstm[b]], ssem[b], add=True)
        pltpu.async_copy(rows_w[b], acc_d.at[dstm[b]], dsem[b], add=True)
        @pl.when(c + 2 < CH)
        def _():
            pltpu.async_copy(hp_hbm.at[src_all.at[c + 2]], rows_g[b],
                             gsem[b])
        return mvec

    for b in range(2):
        pltpu.async_copy(hp_hbm.at[src_all.at[b]], rows_g[b], gsem[b])

    def _pair(g, mvec):
        for b in range(2):
            mvec = _chunk(2 * g + b, b, False, mvec)
        return mvec

    mvec = _chunk(0, 0, True, mvec)
    mvec = _chunk(1, 1, True, mvec)
    mvec = lax.fori_loop(1, CH // 2, _pair, mvec)
    _chunk(CH - 1, 0, False, mvec)

    for b in range(2):
        pltpu.make_async_copy(rows_s[b], acc.at[dstm[b]], ssem[b]).wait()
        pltpu.make_async_copy(rows_w[b], acc_d.at[dstm[b]], dsem[b]).wait()

    plsc.subcore_barrier()
    pltpu.sync_copy(acc.at[pl.ds(sid * RPT, RPT)],
                    out_hbm.at[cid, pl.ds(sid * RPT, RPT)])
    pltpu.sync_copy(acc_d.at[pl.ds(sid * RPT, RPT)],
                    outd_hbm.at[cid, pl.ds(sid * RPT, RPT)])


def kernel(x, edge_index, W1, a_src1, a_dst1, b1, W2, a_src2, a_dst2, b2):
    src = edge_index[0].reshape(NC, NS, CH, K)
    dst = edge_index[1].reshape(NC, NS, CH, K)
    a21 = jnp.stack([a_src1, a_dst1])
    a22 = jnp.stack([a_src2, a_dst2])

    def _as_words(hb):
        return lax.bitcast_convert_type(hb.reshape(N, DP, 2), jnp.int32)

    hp1, sd1 = _tc_prep(x, W1, a21)
    p1, pd1 = _sc_edge(_as_words(hp1), sd1, src, dst)
    hp2, sd2 = _tc_mid(p1, pd1, b1.reshape(1, D), W2, a22)
    p2, pd2 = _sc_edge(_as_words(hp2), sd2, src, dst)
    return _tc_fin(p2, pd2, b2.reshape(1, D))


# speedup vs baseline: 27.7830x; 1.1396x over previous
import functools

import jax
import jax.numpy as jnp
from jax import lax
from jax.experimental import pallas as pl
from jax.experimental.pallas import tpu as pltpu
from jax.experimental.pallas import tpu_sc as plsc

N = 10000
E = 320000
D = 128
DP = 64
DA = 160
NC = 2
NS = 16
NP = 10240
EPT = E // (NC * NS)
K = 80
CH = EPT // K
RPT = NP // NS
ZR = 40
NEG = 0.2
L = 16


def _tc_prep_body(x_ref, w_ref, a2_ref, hp_ref, sd_ref):
    h = jnp.dot(x_ref[...], w_ref[...], preferred_element_type=jnp.float32)
    hp_ref[...] = h.astype(jnp.bfloat16)
    sd_ref[...] = lax.dot_general(
        a2_ref[...], h, (((1,), (1,)), ((), ())),
        preferred_element_type=jnp.float32)


def _tc_prep(x, w, a2):
    return pl.pallas_call(
        _tc_prep_body,
        out_shape=[
            jax.ShapeDtypeStruct((N, D), jnp.bfloat16),
            jax.ShapeDtypeStruct((2, N), jnp.float32),
        ],
    )(x, w, a2)


def _combine(p_ref, b_ref):
    feat = (p_ref[0, :N, :D].astype(jnp.float32)
            + p_ref[1, :N, :D].astype(jnp.float32))
    denom = (p_ref[0, :N, D:D + 1].astype(jnp.float32)
             + p_ref[1, :N, D:D + 1].astype(jnp.float32))
    denom = jnp.where(denom == 0.0, 1.0, denom)
    return feat / denom + b_ref[...]


def _tc_mid_body(p_ref, b_ref, w_ref, a2_ref, hp_ref, sd_ref):
    h1 = jnp.maximum(_combine(p_ref, b_ref), 0.0)
    h2 = jnp.dot(h1, w_ref[...], preferred_element_type=jnp.float32)
    hp_ref[...] = h2.astype(jnp.bfloat16)
    sd_ref[...] = lax.dot_general(
        a2_ref[...], h2, (((1,), (1,)), ((), ())),
        preferred_element_type=jnp.float32)


def _tc_mid(p, b, w, a2):
    return pl.pallas_call(
        _tc_mid_body,
        out_shape=[
            jax.ShapeDtypeStruct((N, D), jnp.bfloat16),
            jax.ShapeDtypeStruct((2, N), jnp.float32),
        ],
    )(p, b, w, a2)


def _tc_fin_body(p_ref, b_ref, o_ref):
    o_ref[...] = _combine(p_ref, b_ref)


def _tc_fin(p, b):
    return pl.pallas_call(
        _tc_fin_body,
        out_shape=jax.ShapeDtypeStruct((N, D), jnp.float32),
    )(p, b)


_mesh = plsc.VectorSubcoreMesh(core_axis_name="c", subcore_axis_name="s", num_cores=NC)


@functools.partial(
    pl.kernel,
    out_type=jax.ShapeDtypeStruct((NC, NP, DA), jnp.bfloat16),
    mesh=_mesh,
    scratch_types=[
        pltpu.VMEM((N,), jnp.float32),
        pltpu.VMEM((N,), jnp.float32),
        pltpu.VMEM((CH, K), jnp.int32),
        pltpu.VMEM((CH, K), jnp.int32),
        pltpu.VMEM((K,), jnp.int32),
        pltpu.VMEM((K,), jnp.int32),
        pltpu.VMEM((K,), jnp.float32),
        pltpu.VMEM((K, DP), jnp.int32),
        pltpu.VMEM((K, DP), jnp.int32),
        pltpu.VMEM((K, DA), jnp.bfloat16),
        pltpu.VMEM((K, DA), jnp.bfloat16),
        pltpu.VMEM((ZR, DA), jnp.bfloat16),
        pltpu.VMEM_SHARED((NP, DA), jnp.bfloat16),
        pltpu.SemaphoreType.DMA,
        pltpu.SemaphoreType.DMA,
        pltpu.SemaphoreType.DMA,
        pltpu.SemaphoreType.DMA,
    ],
    compiler_params=pltpu.CompilerParams(needs_layout_passes=False, use_tc_tiling_on_sc=False),
)
def _sc_edge(hp_hbm, sd_hbm, src_hbm, dst_hbm, out_hbm,
             s_t, d_t, src_all, dst_all, dstm0, dstm1, w_buf,
             rows_g0, rows_g1, rows_s0, rows_s1, zbuf,
             acc, gsem0, gsem1, ssem0, ssem1):
    cid = lax.axis_index("c")
    sid = lax.axis_index("s")
    dstm = (dstm0, dstm1)
    rows_g = (rows_g0, rows_g1)
    rows_s = (rows_s0, rows_s1)
    gsem = (gsem0, gsem1)
    ssem = (ssem0, ssem1)

    pltpu.sync_copy(sd_hbm.at[0], s_t)
    pltpu.sync_copy(sd_hbm.at[1], d_t)
    pltpu.sync_copy(src_hbm.at[cid, sid], src_all)
    pltpu.sync_copy(dst_hbm.at[cid, sid], dst_all)

    zb16 = jnp.zeros((2 * L,), jnp.bfloat16)

    def _zero_row(r, _):
        for j in range(DA // (2 * L)):
            zbuf[r, pl.ds(j * 2 * L, 2 * L)] = zb16
        return 0
    lax.fori_loop(0, ZR, _zero_row, 0)
    for part in range(RPT // ZR):
        pltpu.sync_copy(zbuf, acc.at[pl.ds(sid * RPT + part * ZR, ZR)])

    def _max_body(i, carry):
        ms, md = carry
        ms = jnp.maximum(ms, s_t[pl.ds(i * L, L)])
        md = jnp.maximum(md, d_t[pl.ds(i * L, L)])
        return ms, md
    ninf = jnp.full((L,), -jnp.inf, jnp.float32)
    ms, md = lax.fori_loop(0, N // L, _max_body, (ninf, ninf))
    lanes = lax.iota(jnp.int32, L)
    for sh in (8, 4, 2, 1):
        perm = lanes ^ sh
        ms = jnp.maximum(ms, ms.at[perm].get(mode="promise_in_bounds"))
        md = jnp.maximum(md, md.at[perm].get(mode="promise_in_bounds"))
    mv = ms + md
    mvec = jnp.where(mv > 0.0, mv, NEG * mv)

    onehot = jnp.where(
        lanes == 0,
        jnp.ones((L,), jnp.float32), jnp.zeros((L,), jnp.float32))
    zf = jnp.zeros((L,), jnp.float32)

    plsc.subcore_barrier()

    def _chunk(c, b, first, mvec):
        pltpu.make_async_copy(hp_hbm.at[src_all.at[c]], rows_g[b],
                              gsem[b]).wait()
        if not first:
            pltpu.make_async_copy(rows_s[b], acc.at[dstm[b]], ssem[b]).wait()
        for q in range(K // L):
            si = src_all[c, pl.ds(q * L, L)]
            di = dst_all[c, pl.ds(q * L, L)]
            e = plsc.load_gather(s_t, [si]) + plsc.load_gather(d_t, [di])
            e = jnp.where(e > 0.0, e, NEG * e)
            w_buf[pl.ds(q * L, L)] = jnp.exp(e - mvec)
            dstm[b][pl.ds(q * L, L)] = di
        def _scale(q, _):
            wv16 = w_buf[pl.ds(q * L, L)]
            for u in range(L):
                i = q * L + u
                wv = wv16.at[jnp.full((L,), u, jnp.int32)].get(
                    mode="promise_in_bounds")
                for j in range(DP // L):
                    words = rows_g[b][i, pl.ds(j * L, L)]
                    pair = plsc.unpack(plsc.bitcast(words, jnp.bfloat16),
                                       format=plsc.PackFormat.INTERLEAVED)
                    lo = pair[0].astype(jnp.float32) * wv
                    hi = pair[1].astype(jnp.float32) * wv
                    rows_s[b][i, pl.ds(j * 2 * L, 2 * L)] = plsc.pack(
                        lo, hi, format=plsc.PackFormat.INTERLEAVED)
                rows_s[b][i, pl.ds(D, 2 * L)] = plsc.pack(
                    wv * onehot, zf, format=plsc.PackFormat.INTERLEAVED)
            return 0
        lax.fori_loop(0, K // L, _scale, 0)
        pltpu.async_copy(rows_s[b], acc.at[dstm[b]], ssem[b], add=True)
        @pl.when(c + 2 < CH)
        def _():
            pltpu.async_copy(hp_hbm.at[src_all.at[c + 2]], rows_g[b],
                             gsem[b])
        return mvec

    for b in range(2):
        pltpu.async_copy(hp_hbm.at[src_all.at[b]], rows_g[b], gsem[b])

    def _pair(g, mvec):
        for b in range(2):
            mvec = _chunk(2 * g + b, b, False, mvec)
        return mvec

    mvec = _chunk(0, 0, True, mvec)
    mvec = _chunk(1, 1, True, mvec)
    mvec = lax.fori_loop(1, CH // 2, _pair, mvec)
    _chunk(CH - 1, 0, False, mvec)

    for b in range(2):
        pltpu.make_async_copy(rows_s[b], acc.at[dstm[b]], ssem[b]).wait()

    plsc.subcore_barrier()
    pltpu.sync_copy(acc.at[pl.ds(sid * RPT, RPT)],
                    out_hbm.at[cid, pl.ds(sid * RPT, RPT)])


def kernel(x, edge_index, W1, a_src1, a_dst1, b1, W2, a_src2, a_dst2, b2):
    src = edge_index[0].reshape(NC, NS, CH, K)
    dst = edge_index[1].reshape(NC, NS, CH, K)
    a21 = jnp.stack([a_src1, a_dst1])
    a22 = jnp.stack([a_src2, a_dst2])

    def _as_words(hb):
        return lax.bitcast_convert_type(hb.reshape(N, DP, 2), jnp.int32)

    hp1, sd1 = _tc_prep(x, W1, a21)
    p1 = _sc_edge(_as_words(hp1), sd1, src, dst)
    hp2, sd2 = _tc_mid(p1, b1.reshape(1, D), W2, a22)
    p2 = _sc_edge(_as_words(hp2), sd2, src, dst)
    return _tc_fin(p2, b2.reshape(1, D))


# speedup vs baseline: 29.6414x; 1.0669x over previous
import functools

import jax
import jax.numpy as jnp
from jax import lax
from jax.experimental import pallas as pl
from jax.experimental.pallas import tpu as pltpu
from jax.experimental.pallas import tpu_sc as plsc

N = 10000
E = 320000
D = 128
DP = 64
DA = 160
NC = 2
NS = 16
NP = 10240
EPT = E // (NC * NS)
K = 80
CH = EPT // K
RPT = NP // NS
ZR = 40
NEG = 0.2
L = 16


def _tc_prep_body(x_ref, w_ref, a2_ref, hp_ref, sd_ref):
    h = jnp.dot(x_ref[...], w_ref[...], preferred_element_type=jnp.float32)
    hp_ref[...] = h.astype(jnp.bfloat16)
    sd_ref[...] = lax.dot_general(
        a2_ref[...], h, (((1,), (1,)), ((), ())),
        preferred_element_type=jnp.float32)


def _tc_prep(x, w, a2):
    return pl.pallas_call(
        _tc_prep_body,
        out_shape=[
            jax.ShapeDtypeStruct((N, D), jnp.bfloat16),
            jax.ShapeDtypeStruct((2, N), jnp.float32),
        ],
    )(x, w, a2)


def _combine(p_ref, b_ref):
    feat = (p_ref[0, :N, :D].astype(jnp.float32)
            + p_ref[1, :N, :D].astype(jnp.float32))
    denom = (p_ref[0, :N, D:D + 1].astype(jnp.float32)
             + p_ref[1, :N, D:D + 1].astype(jnp.float32))
    denom = jnp.where(denom == 0.0, 1.0, denom)
    return feat / denom + b_ref[...]


def _tc_mid_body(p_ref, b_ref, w_ref, a2_ref, hp_ref, sd_ref):
    h1 = jnp.maximum(_combine(p_ref, b_ref), 0.0)
    h2 = jnp.dot(h1, w_ref[...], preferred_element_type=jnp.float32)
    hp_ref[...] = h2.astype(jnp.bfloat16)
    sd_ref[...] = lax.dot_general(
        a2_ref[...], h2, (((1,), (1,)), ((), ())),
        preferred_element_type=jnp.float32)


def _tc_mid(p, b, w, a2):
    return pl.pallas_call(
        _tc_mid_body,
        out_shape=[
            jax.ShapeDtypeStruct((N, D), jnp.bfloat16),
            jax.ShapeDtypeStruct((2, N), jnp.float32),
        ],
    )(p, b, w, a2)


def _tc_fin_body(p_ref, b_ref, o_ref):
    o_ref[...] = _combine(p_ref, b_ref)


def _tc_fin(p, b):
    return pl.pallas_call(
        _tc_fin_body,
        out_shape=jax.ShapeDtypeStruct((N, D), jnp.float32),
    )(p, b)


_mesh = plsc.VectorSubcoreMesh(core_axis_name="c", subcore_axis_name="s", num_cores=NC)


@functools.partial(
    pl.kernel,
    out_type=jax.ShapeDtypeStruct((NC, NP, DA), jnp.bfloat16),
    mesh=_mesh,
    scratch_types=[
        pltpu.VMEM((N,), jnp.float32),
        pltpu.VMEM((N,), jnp.float32),
        pltpu.VMEM((CH, K), jnp.int32),
        pltpu.VMEM((CH, K), jnp.int32),
        pltpu.VMEM((K,), jnp.int32),
        pltpu.VMEM((K,), jnp.int32),
        pltpu.VMEM((K,), jnp.float32),
        pltpu.VMEM((K, D), jnp.bfloat16),
        pltpu.VMEM((K, D), jnp.bfloat16),
        pltpu.VMEM((K, DA), jnp.bfloat16),
        pltpu.VMEM((K, DA), jnp.bfloat16),
        pltpu.VMEM((ZR, DA), jnp.bfloat16),
        pltpu.VMEM_SHARED((NP, DA), jnp.bfloat16),
        pltpu.SemaphoreType.DMA,
        pltpu.SemaphoreType.DMA,
        pltpu.SemaphoreType.DMA,
        pltpu.SemaphoreType.DMA,
    ],
    compiler_params=pltpu.CompilerParams(needs_layout_passes=False, use_tc_tiling_on_sc=False),
)
def _sc_edge(hp_hbm, sd_hbm, src_hbm, dst_hbm, out_hbm,
             s_t, d_t, src_all, dst_all, dstm0, dstm1, w_buf,
             rows_g0, rows_g1, rows_s0, rows_s1, zbuf,
             acc, gsem0, gsem1, ssem0, ssem1):
    cid = lax.axis_index("c")
    sid = lax.axis_index("s")
    dstm = (dstm0, dstm1)
    rows_g = (rows_g0, rows_g1)
    rows_s = (rows_s0, rows_s1)
    gsem = (gsem0, gsem1)
    ssem = (ssem0, ssem1)

    pltpu.sync_copy(sd_hbm.at[0], s_t)
    pltpu.sync_copy(sd_hbm.at[1], d_t)
    pltpu.sync_copy(src_hbm.at[cid, sid], src_all)
    pltpu.sync_copy(dst_hbm.at[cid, sid], dst_all)

    zb16 = jnp.zeros((2 * L,), jnp.bfloat16)

    def _zero_row(r, _):
        for j in range(DA // (2 * L)):
            zbuf[r, pl.ds(j * 2 * L, 2 * L)] = zb16
        return 0
    lax.fori_loop(0, ZR, _zero_row, 0)
    for part in range(RPT // ZR):
        pltpu.sync_copy(zbuf, acc.at[pl.ds(sid * RPT + part * ZR, ZR)])

    def _max_body(i, carry):
        ms, md = carry
        ms = jnp.maximum(ms, s_t[pl.ds(i * L, L)])
        md = jnp.maximum(md, d_t[pl.ds(i * L, L)])
        return ms, md
    ninf = jnp.full((L,), -jnp.inf, jnp.float32)
    ms, md = lax.fori_loop(0, N // L, _max_body, (ninf, ninf))
    lanes = lax.iota(jnp.int32, L)
    for sh in (8, 4, 2, 1):
        perm = lanes ^ sh
        ms = jnp.maximum(ms, ms.at[perm].get(mode="promise_in_bounds"))
        md = jnp.maximum(md, md.at[perm].get(mode="promise_in_bounds"))
    mv = ms + md
    mvec = jnp.where(mv > 0.0, mv, NEG * mv)

    onehot = jnp.where(
        lanes == 0,
        jnp.ones((L,), jnp.float32), jnp.zeros((L,), jnp.float32))
    zf = jnp.zeros((L,), jnp.float32)

    plsc.subcore_barrier()

    def _chunk(c, b, first, mvec):
        pltpu.make_async_copy(hp_hbm.at[src_all.at[c]], rows_g[b],
                              gsem[b]).wait()
        if not first:
            pltpu.make_async_copy(rows_s[b], acc.at[dstm[b]], ssem[b]).wait()
        for q in range(K // L):
            si = src_all[c, pl.ds(q * L, L)]
            di = dst_all[c, pl.ds(q * L, L)]
            e = plsc.load_gather(s_t, [si]) + plsc.load_gather(d_t, [di])
            e = jnp.where(e > 0.0, e, NEG * e)
            w_buf[pl.ds(q * L, L)] = jnp.exp(e - mvec)
            dstm[b][pl.ds(q * L, L)] = di
        def _scale(q, _):
            wv16 = w_buf[pl.ds(q * L, L)]
            for u in range(L):
                i = q * L + u
                wv = wv16.at[jnp.full((L,), u, jnp.int32)].get(
                    mode="promise_in_bounds")
                for j in range(DP // L):
                    words = rows_g[b][i, pl.ds(j * 2 * L, 2 * L)]
                    pair = plsc.unpack(words,
                                       format=plsc.PackFormat.INTERLEAVED)
                    lo = pair[0].astype(jnp.float32) * wv
                    hi = pair[1].astype(jnp.float32) * wv
                    rows_s[b][i, pl.ds(j * 2 * L, 2 * L)] = plsc.pack(
                        lo, hi, format=plsc.PackFormat.INTERLEAVED)
                rows_s[b][i, pl.ds(D, 2 * L)] = plsc.pack(
                    wv * onehot, zf, format=plsc.PackFormat.INTERLEAVED)
            return 0
        lax.fori_loop(0, K // L, _scale, 0)
        pltpu.async_copy(rows_s[b], acc.at[dstm[b]], ssem[b], add=True)
        @pl.when(c + 2 < CH)
        def _():
            pltpu.async_copy(hp_hbm.at[src_all.at[c + 2]], rows_g[b],
                             gsem[b])
        return mvec

    for b in range(2):
        pltpu.async_copy(hp_hbm.at[src_all.at[b]], rows_g[b], gsem[b])

    def _pair(g, mvec):
        for b in range(2):
            mvec = _chunk(2 * g + b, b, False, mvec)
        return mvec

    mvec = _chunk(0, 0, True, mvec)
    mvec = _chunk(1, 1, True, mvec)
    mvec = lax.fori_loop(1, CH // 2, _pair, mvec)
    _chunk(CH - 1, 0, False, mvec)

    for b in range(2):
        pltpu.make_async_copy(rows_s[b], acc.at[dstm[b]], ssem[b]).wait()

    plsc.subcore_barrier()
    pltpu.sync_copy(acc.at[pl.ds(sid * RPT, RPT)],
                    out_hbm.at[cid, pl.ds(sid * RPT, RPT)])


def kernel(x, edge_index, W1, a_src1, a_dst1, b1, W2, a_src2, a_dst2, b2):
    src = edge_index[0].reshape(NC, NS, CH, K)
    dst = edge_index[1].reshape(NC, NS, CH, K)
    a21 = jnp.stack([a_src1, a_dst1])
    a22 = jnp.stack([a_src2, a_dst2])

    hp1, sd1 = _tc_prep(x, W1, a21)
    p1 = _sc_edge(hp1, sd1, src, dst)
    hp2, sd2 = _tc_mid(p1, b1.reshape(1, D), W2, a22)
    p2 = _sc_edge(hp2, sd2, src, dst)
    return _tc_fin(p2, b2.reshape(1, D))
